# baseline (speedup 1.0000x reference)
import jax
import jax.numpy as jnp
from jax import lax
from jax.experimental import pallas as pl
from jax.experimental.pallas import tpu as pltpu
from jax.experimental.pallas import tpu_sc as plsc

B, S, E = 4, 2048, 1024
NW = 32
ROWS_PER_W = S // NW
C = 4
NCHUNK = ROWS_PER_W // C
NBUF = 6
LANES = 16
VPR = E // LANES
NVEC = C * VPR
SCALE = 32.0


def _sc_body(in_hbm, tab_hbm, out_hbm, tbuf, ibuf, sem_in, sem_out):
    wid = lax.axis_index("s") * 2 + lax.axis_index("c")
    s0 = wid * ROWS_PER_W

    def in_copies(c):
        slot = lax.rem(c, NBUF)
        row0 = s0 + c * C
        return (
            pltpu.make_async_copy(
                tab_hbm.at[pl.ds(row0, C), :], tbuf.at[slot],
                sem_in.at[slot]),
            pltpu.make_async_copy(
                in_hbm.at[:, pl.ds(row0, C), :], ibuf.at[slot],
                sem_in.at[slot]),
        )

    def out_copy(c):
        slot = lax.rem(c, NBUF)
        row0 = s0 + c * C
        return pltpu.make_async_copy(
            ibuf.at[slot], out_hbm.at[:, pl.ds(row0, C), :],
            sem_out.at[slot])

    for c in range(NBUF - 1):
        for d in in_copies(jnp.int32(c)):
            d.start()

    def body(c, carry):
        @pl.when(c >= 1)
        def _():
            out_copy(c - 1).wait()

        @pl.when(c + NBUF - 1 < NCHUNK)
        def _():
            for d in in_copies(c + NBUF - 1):
                d.start()

        for d in in_copies(c):
            d.wait()

        slot = lax.rem(c, NBUF)

        @plsc.parallel_loop(0, NVEC, unroll=8)
        def _(i):
            r = i >> 6
            off = (i & (VPR - 1)) * LANES
            tv = tbuf[slot, r, pl.ds(off, LANES)] * SCALE
            for b in range(B):
                ibuf[slot, b, r, pl.ds(off, LANES)] = (
                    ibuf[slot, b, r, pl.ds(off, LANES)] + tv)

        out_copy(c).start()
        return carry

    lax.fori_loop(0, NCHUNK, body, jnp.int32(0))
    out_copy(jnp.int32(NCHUNK - 1)).wait()


_sc_call = pl.kernel(
    _sc_body,
    out_type=jax.ShapeDtypeStruct((B, S, E), jnp.float32),
    mesh=plsc.VectorSubcoreMesh(core_axis_name="c", subcore_axis_name="s"),
    scratch_types=[
        pltpu.VMEM((NBUF, C, E), jnp.float32),
        pltpu.VMEM((NBUF, B, C, E), jnp.float32),
        pltpu.SemaphoreType.DMA((NBUF,)),
        pltpu.SemaphoreType.DMA((NBUF,)),
    ],
)


@jax.jit
def kernel(inputs, embedding_table, pos_encoding):
    del pos_encoding
    return _sc_call(inputs, embedding_table)
